# Initial kernel scaffold; baseline (speedup 1.0000x reference)
import jax
import jax.numpy as jnp
from jax.experimental import pallas as pl


def kernel(x, edge_index, W1, a_src1, a_dst1, b1, W2, a_src2, a_dst2, b2):
    raise NotImplementedError("write your pallas kernel here")


# speedup vs baseline: 46.5992x; 46.5992x over previous
import functools

import jax
import jax.numpy as jnp
from jax import lax
from jax.experimental import pallas as pl
from jax.experimental.pallas import tpu as pltpu
from jax.experimental.pallas import tpu_sc as plsc

N = 10000
E = 320000
D_IN = 128
HEADS = 8
HID = 16

NC = 2
NS = 16
NW = NC * NS
NPAD = 10240
EPAD = 327680
EW = EPAD // NW
S = 128
CS = EW // S
ROWS_T = NPAD // NS

_SLOPE = 0.2


def _lrelu(v):
    return jnp.maximum(v, _SLOPE * v)


def _take16(v, idx):
    dn = lax.GatherDimensionNumbers(
        offset_dims=(), collapsed_slice_dims=(0,), start_index_map=(0,))
    return lax.gather(v, idx[:, None], dn, slice_sizes=(1,),
                      mode=lax.GatherScatterMode.PROMISE_IN_BOUNDS)


def _tc1_body(x_ref, w1_ref, a_ref, h_ref, comb_ref, maxc_ref):
    h = jnp.dot(x_ref[...], w1_ref[...], preferred_element_type=jnp.float32)
    h_ref[...] = h
    comb = jnp.dot(h, a_ref[...], preferred_element_type=jnp.float32)
    comb_ref[...] = comb
    maxc_ref[...] = jnp.max(comb, axis=0, keepdims=True)


def _tc1(xp, W1, A):
    return pl.pallas_call(
        _tc1_body,
        out_shape=(
            jax.ShapeDtypeStruct((NPAD, 128), jnp.float32),
            jax.ShapeDtypeStruct((NPAD, 16), jnp.float32),
            jax.ShapeDtypeStruct((1, 16), jnp.float32),
        ),
    )(xp, W1, A)


def _sc1_body(src_hbm, dst_hbm, comb_hbm, h_hbm, m_hbm, zn_hbm, znd_hbm,
              nump_hbm, denp_hbm,
              srci_v, dsti_v, ag_v, bg_v, wg_v, hg_v, m_v,
              num_sp, den_sp):
    cid = lax.axis_index("c")
    sid = lax.axis_index("s")
    wid = sid * NC + cid

    r0 = sid * ROWS_T
    pltpu.sync_copy(zn_hbm.at[pl.ds(r0, ROWS_T), :], num_sp.at[pl.ds(r0, ROWS_T), :])
    pltpu.sync_copy(znd_hbm.at[pl.ds(r0, ROWS_T), :], den_sp.at[pl.ds(r0, ROWS_T), :])
    pltpu.sync_copy(m_hbm, m_v)
    plsc.subcore_barrier()

    mvec = m_v[...]
    pidx = (lax.iota(jnp.int32, 16) & 7) + 8

    row0 = wid * CS

    def chunk(j, carry):
        pltpu.sync_copy(src_hbm.at[row0 + j], srci_v.at[0])
        pltpu.sync_copy(dst_hbm.at[row0 + j], dsti_v.at[0])
        pltpu.sync_copy(comb_hbm.at[srci_v.at[0]], ag_v)
        pltpu.sync_copy(comb_hbm.at[dsti_v.at[0]], bg_v)
        pltpu.sync_copy(h_hbm.at[srci_v.at[0]], hg_v)

        def edge(r, c2):
            a = ag_v[r, :]
            b = bg_v[r, :]
            e = a + _take16(b, pidx)
            w = jnp.exp(_lrelu(e) - mvec)
            wg_v[r, :] = w
            for k in range(HEADS):
                hk = hg_v[r, pl.ds(16 * k, 16)]
                wk = _take16(w, jnp.full((16,), k, jnp.int32))
                hg_v[r, pl.ds(16 * k, 16)] = hk * wk
            return c2

        lax.fori_loop(0, S, edge, 0)

        pltpu.sync_copy(hg_v, num_sp.at[dsti_v.at[0]], add=True)
        pltpu.sync_copy(wg_v, den_sp.at[dsti_v.at[0]], add=True)
        return carry

    lax.fori_loop(0, CS, chunk, 0)

    plsc.subcore_barrier()
    pltpu.sync_copy(num_sp.at[pl.ds(r0, ROWS_T), :],
                    nump_hbm.at[cid, pl.ds(r0, ROWS_T), :])
    pltpu.sync_copy(den_sp.at[pl.ds(r0, ROWS_T), :],
                    denp_hbm.at[cid, pl.ds(r0, ROWS_T), :])


def _sc1(src2d, dst2d, comb, h, m16, zn, znd):
    mesh = plsc.VectorSubcoreMesh(core_axis_name="c", subcore_axis_name="s")
    f = pl.kernel(
        _sc1_body,
        out_type=(
            jax.ShapeDtypeStruct((NC, NPAD, 128), jnp.float32),
            jax.ShapeDtypeStruct((NC, NPAD, 16), jnp.float32),
        ),
        mesh=mesh,
        compiler_params=pltpu.CompilerParams(use_tc_tiling_on_sc=False),
        scratch_types=[
            pltpu.VMEM((1, S), jnp.int32),
            pltpu.VMEM((1, S), jnp.int32),
            pltpu.VMEM((S, 16), jnp.float32),
            pltpu.VMEM((S, 16), jnp.float32),
            pltpu.VMEM((S, 16), jnp.float32),
            pltpu.VMEM((S, 128), jnp.float32),
            pltpu.VMEM((16,), jnp.float32),
            pltpu.VMEM_SHARED((NPAD, 128), jnp.float32),
            pltpu.VMEM_SHARED((NPAD, 16), jnp.float32),
        ],
    )
    return f(src2d, dst2d, comb, h, m16, zn, znd)


def _tc2_body(nump_ref, denp_ref, r_ref, b1_ref, w2_ref,
              h2_ref, mx_ref, mn_ref):
    num = nump_ref[0] + nump_ref[1]
    den = denp_ref[0] + denp_ref[1]
    denrep = jnp.dot(den, r_ref[...], preferred_element_type=jnp.float32)
    h1 = jax.nn.relu(num / (denrep + 1e-16) + b1_ref[...])
    rows = lax.broadcasted_iota(jnp.int32, (NPAD, 128), 0)
    h1 = jnp.where(rows < N, h1, 0.0)
    h2 = jnp.dot(h1, w2_ref[...], preferred_element_type=jnp.float32)
    h2_ref[...] = h2
    rows1 = lax.broadcasted_iota(jnp.int32, (NPAD, 1), 0)
    mx_ref[...] = jnp.max(jnp.where(rows1 < N, h2, -jnp.inf), axis=0,
                          keepdims=True)
    mn_ref[...] = jnp.min(jnp.where(rows1 < N, h2, jnp.inf), axis=0,
                          keepdims=True)


def _tc2(nump, denp, R, b1row, W2):
    return pl.pallas_call(
        _tc2_body,
        out_shape=(
            jax.ShapeDtypeStruct((NPAD, 1), jnp.float32),
            jax.ShapeDtypeStruct((1, 1), jnp.float32),
            jax.ShapeDtypeStruct((1, 1), jnp.float32),
        ),
    )(nump, denp, R, b1row, W2)


def _sc2_body(src_hbm, dst_hbm, h2_hbm, c2_hbm, z1_hbm,
              num2_hbm, den2_hbm,
              srcb_v, dstb_v, h2_v, c2_v, wbuf_v, pbuf_v,
              num_sp, den_sp):
    cid = lax.axis_index("c")
    sid = lax.axis_index("s")
    wid = sid * NC + cid

    r0 = sid * ROWS_T
    pltpu.sync_copy(z1_hbm.at[pl.ds(r0, ROWS_T)], num_sp.at[pl.ds(r0, ROWS_T)])
    pltpu.sync_copy(z1_hbm.at[pl.ds(r0, ROWS_T)], den_sp.at[pl.ds(r0, ROWS_T)])
    pltpu.sync_copy(h2_hbm, h2_v)
    pltpu.sync_copy(c2_hbm, c2_v)
    row0 = wid * CS
    pltpu.sync_copy(src_hbm.at[pl.ds(row0, CS), :], srcb_v)
    pltpu.sync_copy(dst_hbm.at[pl.ds(row0, CS), :], dstb_v)
    plsc.subcore_barrier()

    c2 = c2_v[...]
    as2 = _take16(c2, jnp.full((16,), 0, jnp.int32))
    ad2 = _take16(c2, jnp.full((16,), 1, jnp.int32))
    m2 = _take16(c2, jnp.full((16,), 2, jnp.int32))

    def chunk(j, carry):
        def group(g, c2_):
            si = srcb_v[j, pl.ds(16 * g, 16)]
            di = dstb_v[j, pl.ds(16 * g, 16)]
            hs = plsc.load_gather(h2_v, [si])
            hd = plsc.load_gather(h2_v, [di])
            w = jnp.exp(_lrelu(as2 * hs + ad2 * hd) - m2)
            wbuf_v[pl.ds(16 * g, 16)] = w
            pbuf_v[pl.ds(16 * g, 16)] = w * hs
            return c2_

        lax.fori_loop(0, S // 16, group, 0)
        pltpu.sync_copy(wbuf_v, den_sp.at[dstb_v.at[j]], add=True)
        pltpu.sync_copy(pbuf_v, num_sp.at[dstb_v.at[j]], add=True)
        return carry

    lax.fori_loop(0, CS, chunk, 0)

    plsc.subcore_barrier()
    pltpu.sync_copy(num_sp.at[pl.ds(r0, ROWS_T)],
                    num2_hbm.at[cid, pl.ds(r0, ROWS_T)])
    pltpu.sync_copy(den_sp.at[pl.ds(r0, ROWS_T)],
                    den2_hbm.at[cid, pl.ds(r0, ROWS_T)])


def _sc2(src2d, dst2d, h2v, c2v, z1):
    mesh = plsc.VectorSubcoreMesh(core_axis_name="c", subcore_axis_name="s")
    f = pl.kernel(
        _sc2_body,
        out_type=(
            jax.ShapeDtypeStruct((NC, NPAD), jnp.float32),
            jax.ShapeDtypeStruct((NC, NPAD), jnp.float32),
        ),
        mesh=mesh,
        compiler_params=pltpu.CompilerParams(needs_layout_passes=False),
        scratch_types=[
            pltpu.VMEM((CS, S), jnp.int32),
            pltpu.VMEM((CS, S), jnp.int32),
            pltpu.VMEM((NPAD,), jnp.float32),
            pltpu.VMEM((16,), jnp.float32),
            pltpu.VMEM((S,), jnp.float32),
            pltpu.VMEM((S,), jnp.float32),
            pltpu.VMEM_SHARED((NPAD,), jnp.float32),
            pltpu.VMEM_SHARED((NPAD,), jnp.float32),
        ],
    )
    return f(src2d, dst2d, h2v, c2v, z1)


def _tc3_body(n2_ref, d2_ref, b2_ref, out_ref):
    num = n2_ref[0] + n2_ref[1]
    den = d2_ref[0] + d2_ref[1]
    out_ref[...] = num / (den + 1e-16) + b2_ref[0, 0]


def _tc3(n2, d2, b2):
    return pl.pallas_call(
        _tc3_body,
        out_shape=jax.ShapeDtypeStruct((NPAD // 128, 128), jnp.float32),
    )(n2, d2, b2)


def kernel(x, edge_index, W1, a_src1, a_dst1, b1, W2, a_src2, a_dst2, b2):
    f32 = jnp.float32
    x = x.astype(f32)
    xp = jnp.pad(x, ((0, NPAD - N), (0, 0)))

    eye = jnp.eye(HEADS, dtype=f32)
    A_src = (eye[:, None, :] * a_src1.astype(f32)[:, :, None]).reshape(128, 8)
    A_dst = (eye[:, None, :] * a_dst1.astype(f32)[:, :, None]).reshape(128, 8)
    A = jnp.concatenate([A_src, A_dst], axis=1)

    src = edge_index[0].astype(jnp.int32)
    dst = edge_index[1].astype(jnp.int32)
    src2d = jnp.pad(src, (0, EPAD - E), constant_values=N).reshape(EPAD // S, S)
    dst2d = jnp.pad(dst, (0, EPAD - E), constant_values=N).reshape(EPAD // S, S)

    h, comb, maxc = _tc1(xp, W1.astype(f32), A)

    m8 = _lrelu(maxc[0, :8] + maxc[0, 8:])
    m16 = jnp.concatenate([m8, m8])

    zn = jnp.zeros((NPAD, 128), f32)
    znd = jnp.zeros((NPAD, 16), f32)
    nump, denp = _sc1(src2d, dst2d, comb, h, m16, zn, znd)

    ch = jnp.arange(128) // 16
    R = (jnp.arange(16)[:, None] == ch[None, :]).astype(f32)
    b1row = b1.astype(f32).reshape(1, 128)
    h2c, mx, mn = _tc2(nump, denp, R, b1row, W2.astype(f32))

    as2 = a_src2.astype(f32)[0, 0]
    ad2 = a_dst2.astype(f32)[0, 0]
    mxs, mns = mx[0, 0], mn[0, 0]
    m2 = _lrelu(jnp.maximum(as2 * mxs, as2 * mns)
                + jnp.maximum(ad2 * mxs, ad2 * mns))
    c2v = jnp.stack([as2, ad2, m2])
    c2v = jnp.pad(c2v, (0, 13))

    h2v = h2c.reshape(NPAD)
    z1 = jnp.zeros((NPAD,), f32)
    num2, den2 = _sc2(src2d, dst2d, h2v, c2v, z1)

    out = _tc3(num2.reshape(NC, NPAD // 128, 128),
               den2.reshape(NC, NPAD // 128, 128),
               b2.astype(f32).reshape(1, 1))
    return out.reshape(NPAD)[:N, None]
